# Initial kernel scaffold; baseline (speedup 1.0000x reference)
import functools
import math

import jax
import jax.numpy as jnp
from jax.experimental import pallas as pl
from jax.experimental.pallas import tpu as pltpu
from jax.experimental.pallas import tpu_sc as plsc

EMB_DIM = 32
SCALE = math.sqrt(float(EMB_DIM))
LANES = 16
WINDOW = 128


def _build_kernel(n_tokens):
    mesh = plsc.VectorSubcoreMesh(core_axis_name="c", subcore_axis_name="s")

    @functools.partial(
        pl.kernel,
        out_type=jax.ShapeDtypeStruct((n_tokens, EMB_DIM), jnp.float32),
        mesh=mesh,
    )
    def emb_kernel(table_hbm, idx_hbm, out_hbm):
        def body(i_vmem, o_vmem):
            pltpu.sync_copy(table_hbm.at[i_vmem.at[0]], o_vmem)

            zeros_i = jnp.zeros((LANES,), jnp.int32)
            zero_f = jnp.zeros((LANES,), jnp.float32)
            scale_f = jnp.full((LANES,), SCALE, jnp.float32)

            @pl.loop(0, WINDOW)
            def _(r):
                bidx = plsc.load_gather(
                    i_vmem, [zeros_i, jnp.full((LANES,), r, jnp.int32)]
                )
                svec = jnp.where(bidx == 1, zero_f, scale_f)
                o_vmem[r, pl.ds(0, LANES)] = o_vmem[r, pl.ds(0, LANES)] * svec
                o_vmem[r, pl.ds(LANES, LANES)] = (
                    o_vmem[r, pl.ds(LANES, LANES)] * svec
                )

        pltpu.emit_pipeline(
            body,
            grid=(n_tokens // WINDOW,),
            in_specs=[pl.BlockSpec((1, WINDOW), lambda i: (0, i))],
            out_specs=[pl.BlockSpec((WINDOW, EMB_DIM), lambda i: (i, 0))],
            core_axis_name=("c", "s"),
            dimension_semantics=(pltpu.PARALLEL,),
        )(idx_hbm, out_hbm)

    return emb_kernel


def kernel(tokens, table):
    b, l = tokens.shape
    n = b * l
    idx = tokens.reshape(1, n).astype(jnp.int32)
    out = _build_kernel(n)(table, idx)
    return out.reshape(b, l, EMB_DIM)


# speedup vs baseline: 1.2136x; 1.2136x over previous
import dataclasses
import functools
import math

import jax
import jax.numpy as jnp
from jax.experimental import pallas as pl
from jax.experimental.pallas import tpu as pltpu
from jax.experimental.pallas import tpu_sc as plsc

EMB_DIM = 32
SCALE = math.sqrt(float(EMB_DIM))
LANES = 16
WINDOW = 128


def _build_kernel(n_tokens):
    mesh = plsc.VectorSubcoreMesh(core_axis_name="c", subcore_axis_name="s")
    cp = pltpu.CompilerParams(
        needs_layout_passes=False, use_tc_tiling_on_sc=False
    )

    @functools.partial(
        pl.kernel,
        out_type=jax.ShapeDtypeStruct((n_tokens, EMB_DIM), jnp.float32),
        mesh=mesh,
        compiler_params=cp,
    )
    def emb_kernel(table_hbm, idx_hbm, out_hbm):
        def body(i_vmem, o_vmem):
            pltpu.sync_copy(table_hbm.at[i_vmem.at[0]], o_vmem)

            zeros_i = jnp.zeros((LANES,), jnp.int32)
            zero_f = jnp.zeros((LANES,), jnp.float32)
            scale_f = jnp.full((LANES,), SCALE, jnp.float32)

            @pl.loop(0, WINDOW)
            def _(r):
                bidx = plsc.load_gather(
                    i_vmem, [zeros_i, jnp.full((LANES,), r, jnp.int32)]
                )
                svec = jnp.where(bidx == 1, zero_f, scale_f)
                o_vmem[r, pl.ds(0, LANES)] = o_vmem[r, pl.ds(0, LANES)] * svec
                o_vmem[r, pl.ds(LANES, LANES)] = (
                    o_vmem[r, pl.ds(LANES, LANES)] * svec
                )

        pltpu.emit_pipeline(
            body,
            grid=(n_tokens // WINDOW,),
            in_specs=[pl.BlockSpec((1, WINDOW), lambda i: (0, i))],
            out_specs=[pl.BlockSpec((WINDOW, EMB_DIM), lambda i: (i, 0))],
            core_axis_name=("c", "s"),
            dimension_semantics=(pltpu.PARALLEL,),
        )(idx_hbm, out_hbm)

    return emb_kernel


def kernel(tokens, table):
    b, l = tokens.shape
    n = b * l
    idx = tokens.reshape(1, n).astype(jnp.int32)
    out = _build_kernel(n)(table, idx)
    return out.reshape(b, l, EMB_DIM)


# speedup vs baseline: 1.2911x; 1.0638x over previous
import dataclasses
import functools
import math

import jax
import jax.numpy as jnp
from jax.experimental import pallas as pl
from jax.experimental.pallas import tpu as pltpu
from jax.experimental.pallas import tpu_sc as plsc

EMB_DIM = 32
SCALE = math.sqrt(float(EMB_DIM))
LANES = 16
WINDOW = 512


def _build_kernel(n_tokens):
    mesh = plsc.VectorSubcoreMesh(core_axis_name="c", subcore_axis_name="s")
    cp = pltpu.CompilerParams(
        needs_layout_passes=False, use_tc_tiling_on_sc=False
    )

    @functools.partial(
        pl.kernel,
        out_type=jax.ShapeDtypeStruct((n_tokens, EMB_DIM), jnp.float32),
        mesh=mesh,
        compiler_params=cp,
    )
    def emb_kernel(table_hbm, idx_hbm, out_hbm):
        def body(i_vmem, o_vmem):
            pltpu.sync_copy(table_hbm.at[i_vmem.at[0]], o_vmem)

            zeros_i = jnp.zeros((LANES,), jnp.int32)
            zero_f = jnp.zeros((LANES,), jnp.float32)
            scale_f = jnp.full((LANES,), SCALE, jnp.float32)

            @pl.loop(0, WINDOW)
            def _(r):
                bidx = plsc.load_gather(
                    i_vmem, [zeros_i, jnp.full((LANES,), r, jnp.int32)]
                )
                svec = jnp.where(bidx == 1, zero_f, scale_f)
                o_vmem[r, pl.ds(0, LANES)] = o_vmem[r, pl.ds(0, LANES)] * svec
                o_vmem[r, pl.ds(LANES, LANES)] = (
                    o_vmem[r, pl.ds(LANES, LANES)] * svec
                )

        pltpu.emit_pipeline(
            body,
            grid=(n_tokens // WINDOW,),
            in_specs=[pl.BlockSpec((1, WINDOW), lambda i: (0, i))],
            out_specs=[pl.BlockSpec((WINDOW, EMB_DIM), lambda i: (i, 0))],
            core_axis_name=("c", "s"),
            dimension_semantics=(pltpu.PARALLEL,),
        )(idx_hbm, out_hbm)

    return emb_kernel


def kernel(tokens, table):
    b, l = tokens.shape
    n = b * l
    idx = tokens.reshape(1, n).astype(jnp.int32)
    out = _build_kernel(n)(table, idx)
    return out.reshape(b, l, EMB_DIM)
